# baseline (speedup 1.0000x reference)
import functools
import math

import jax
import jax.numpy as jnp
from jax import lax
from jax.experimental import pallas as pl
from jax.experimental.pallas import tpu as pltpu
from jax.experimental.pallas import tpu_sc as plsc

_INFO = plsc.get_sparse_core_info()
_NC, _NS, _L = _INFO.num_cores, _INFO.num_subcores, _INFO.num_lanes
_NW = _NC * _NS

_CB = 8


def _pad8(n):
    return (n + 7) // 8 * 8


@functools.partial(jax.jit, static_argnums=(2,))
def _embed_lookup(x, table, rows_per_w):
    b, h = x.shape
    _, d = table.shape
    scale = math.sqrt(d)
    n_chunks = rows_per_w // _CB
    hp = _pad8(h)
    n_idx = _CB * hp
    mesh = plsc.VectorSubcoreMesh(core_axis_name="c", subcore_axis_name="s")

    @functools.partial(
        pl.kernel,
        out_type=jax.ShapeDtypeStruct((b, h, d), jnp.float32),
        mesh=mesh,
        scratch_types=[
            pltpu.VMEM((n_idx,), jnp.int32),
            pltpu.VMEM((n_idx, d), jnp.float32),
            pltpu.SemaphoreType.DMA,
            pltpu.SemaphoreType.DMA,
        ],
        compiler_params=pltpu.CompilerParams(use_tc_tiling_on_sc=False),
    )
    def body(x_hbm, table_hbm, out_hbm, idx_v, rows_v, sem, wsem):
        wid = lax.axis_index("s") * _NC + lax.axis_index("c")
        base = wid * rows_per_w

        zeros = jnp.zeros((_L,), jnp.int32)

        @pl.loop(0, n_idx // _L)
        def _z(k):
            idx_v[pl.ds(k * _L, _L)] = zeros

        @pl.loop(0, n_chunks)
        def _chunk(i):
            b0 = base + i * _CB
            for r in range(_CB):
                pltpu.async_copy(
                    x_hbm.at[b0 + r], idx_v.at[pl.ds(r * hp, h)], sem
                )
            for r in range(_CB):
                pltpu.make_async_copy(
                    x_hbm.at[b0 + r], idx_v.at[pl.ds(r * hp, h)], sem
                ).wait()
            pltpu.async_copy(table_hbm.at[idx_v], rows_v, sem).wait()

            @pl.loop(0, n_idx, unroll=2)
            def _row(r):
                for j in range(d // _L):
                    sl = pl.ds(j * _L, _L)
                    rows_v[r, sl] = rows_v[r, sl] * scale

            for r in range(_CB):
                pltpu.async_copy(
                    rows_v.at[pl.ds(r * hp, h)], out_hbm.at[b0 + r], wsem
                )
            for r in range(_CB):
                pltpu.make_async_copy(
                    rows_v.at[pl.ds(r * hp, h)], out_hbm.at[b0 + r], wsem
                ).wait()

    return body(x, table)


def kernel(x, table):
    b, h = x.shape
    return _embed_lookup(x, table, b // _NW)


# speedup vs baseline: 1.7393x; 1.7393x over previous
import functools
import math

import jax
import jax.numpy as jnp
from jax import lax
from jax.experimental import pallas as pl
from jax.experimental.pallas import tpu as pltpu
from jax.experimental.pallas import tpu_sc as plsc

_INFO = plsc.get_sparse_core_info()
_NC, _NS, _L = _INFO.num_cores, _INFO.num_subcores, _INFO.num_lanes
_NW = _NC * _NS


@jax.jit
def _embed_lookup(x, table):
    b, h = x.shape
    _, d = table.shape
    scale = math.sqrt(d)
    bw = b // _NW
    nbk = bw // 128
    n_units = h * nbk
    hp = (h + 7) // 8 * 8
    et = d // 8
    mesh = plsc.VectorSubcoreMesh(core_axis_name="c", subcore_axis_name="s")

    @functools.partial(
        pl.kernel,
        out_type=jax.ShapeDtypeStruct((h, et, b // 128, 8, 128), jnp.float32),
        mesh=mesh,
        scratch_types=[
            pltpu.VMEM((bw, h), jnp.int32),
            pltpu.VMEM((h * bw,), jnp.int32),
            pltpu.VMEM((128, d), jnp.float32),
            pltpu.VMEM((128, d), jnp.float32),
            pltpu.VMEM((et, 8, 128), jnp.float32),
            pltpu.VMEM((et, 8, 128), jnp.float32),
            pltpu.VMEM((et, 8, 128), jnp.float32),
            pltpu.VMEM((et, 8, 128), jnp.float32),
            pltpu.SemaphoreType.DMA,
            pltpu.SemaphoreType.DMA,
            pltpu.SemaphoreType.DMA,
            pltpu.SemaphoreType.DMA,
            pltpu.SemaphoreType.DMA,
            pltpu.SemaphoreType.DMA,
        ],
        compiler_params=pltpu.CompilerParams(
            use_tc_tiling_on_sc=False, needs_layout_passes=False
        ),
    )
    def body(x_hbm, table_hbm, out_hbm, xs, idxb, r0_v, r1_v,
             o0, o1, o2, o3, g0, g1, w0, w1, w2, w3):
        wid = lax.axis_index("s") * _NC + lax.axis_index("c")
        b0 = wid * bw
        iota = lax.iota(jnp.int32, _L)
        rows = (r0_v, r1_v)
        gsems = (g0, g1)
        obs = (o0, o1, o2, o3)
        wsems = (w0, w1, w2, w3)

        pltpu.sync_copy(x_hbm.at[pl.ds(b0, bw)], xs)

        n_full = h // _L
        tail_lo = n_full * _L
        tail_mask = iota < (h - tail_lo)

        @pl.loop(0, bw)
        def _stage(r):
            rvec = iota * 0 + r
            for j in range(n_full):
                vals = xs[r, pl.ds(j * _L, _L)]
                pos = (j * _L + iota) * bw + r
                plsc.store_scatter(idxb, [pos], vals)
            if h % _L:
                vals = plsc.load_gather(
                    xs, [rvec, tail_lo + iota], mask=tail_mask
                )
                pos = (tail_lo + iota) * bw + r
                plsc.store_scatter(idxb, [pos], vals, mask=tail_mask)

        def g_desc(u, par2):
            off = (u // nbk) * bw + (u % nbk) * 128
            return pltpu.make_async_copy(
                table_hbm.at[idxb.at[pl.ds(off, 128)]], rows[par2], gsems[par2]
            )

        g_desc(0, 0).start()

        @pl.loop(0, n_units // 4)
        def _it(it):
            for par in range(4):
                u = it * 4 + par
                rbuf = rows[par % 2]
                ob = obs[par]
                g_desc(u, par % 2).wait()
                if par < 3:
                    g_desc(u + 1, (par + 1) % 2).start()
                else:
                    @pl.when(it < n_units // 4 - 1)
                    def _():
                        g_desc(u + 1, (par + 1) % 2).start()

                @pl.when(it >= 1)
                def _():
                    for ej in range(et):
                        pltpu.make_async_copy(
                            ob.at[ej],
                            out_hbm.at[it - 1, ej, wid * nbk + par],
                            wsems[par],
                        ).wait()

                @pl.loop(0, d)
                def _col(v):
                    col = iota * 0 + v
                    ej, s = v // 8, v % 8
                    for r0 in range(0, 128, _L):
                        vals = plsc.load_gather(rbuf, [r0 + iota, col])
                        ob[ej, s, pl.ds(r0, _L)] = vals * scale

                for ej in range(et):
                    pltpu.async_copy(
                        ob.at[ej], out_hbm.at[it, ej, wid * nbk + par],
                        wsems[par],
                    )

        for par in range(4):
            for ej in range(et):
                pltpu.make_async_copy(
                    obs[par].at[ej],
                    out_hbm.at[n_units // 4 - 1, ej, wid * nbk + par],
                    wsems[par],
                ).wait()

    raw = body(x, table)
    return raw.transpose(2, 4, 0, 1, 3).reshape(b, h, d)


def kernel(x, table):
    return _embed_lookup(x, table)


# speedup vs baseline: 2.6818x; 1.5419x over previous
import functools
import math

import jax
import jax.numpy as jnp
from jax import lax
from jax.experimental import pallas as pl
from jax.experimental.pallas import tpu as pltpu
from jax.experimental.pallas import tpu_sc as plsc

_INFO = plsc.get_sparse_core_info()
_NC, _NS, _L = _INFO.num_cores, _INFO.num_subcores, _INFO.num_lanes
_NW = _NC * _NS


@jax.jit
def _embed_lookup(x, table):
    b, h = x.shape
    _, d = table.shape
    scale = math.sqrt(d)
    bw = b // _NW
    nbk = bw // 128
    n_units = h * nbk
    hp = (h + 7) // 8 * 8
    et = d // 8
    mesh = plsc.VectorSubcoreMesh(core_axis_name="c", subcore_axis_name="s")

    @functools.partial(
        pl.kernel,
        out_type=jax.ShapeDtypeStruct((h, et, b // 128, 8, 128), jnp.float32),
        mesh=mesh,
        scratch_types=[
            pltpu.VMEM((bw, h), jnp.int32),
            pltpu.VMEM((h * bw,), jnp.int32),
            pltpu.VMEM((128, d), jnp.float32),
            pltpu.VMEM((128, d), jnp.float32),
            pltpu.VMEM((et, 8, 128), jnp.float32),
            pltpu.VMEM((et, 8, 128), jnp.float32),
            pltpu.VMEM((et, 8, 128), jnp.float32),
            pltpu.VMEM((et, 8, 128), jnp.float32),
            pltpu.SemaphoreType.DMA,
            pltpu.SemaphoreType.DMA,
            pltpu.SemaphoreType.DMA,
            pltpu.SemaphoreType.DMA,
            pltpu.SemaphoreType.DMA,
            pltpu.SemaphoreType.DMA,
        ],
        compiler_params=pltpu.CompilerParams(
            use_tc_tiling_on_sc=False, needs_layout_passes=False
        ),
    )
    def body(x_hbm, table_hbm, out_hbm, xs, idxb, r0_v, r1_v,
             o0, o1, o2, o3, g0, g1, w0, w1, w2, w3):
        wid = lax.axis_index("s") * _NC + lax.axis_index("c")
        b0 = wid * bw
        iota = lax.iota(jnp.int32, _L)
        rows = (r0_v, r1_v)
        gsems = (g0, g1)
        obs = (o0, o1, o2, o3)
        wsems = (w0, w1, w2, w3)

        pltpu.sync_copy(x_hbm.at[pl.ds(b0, bw)], xs)

        n_full = h // _L
        tail_lo = n_full * _L
        tail_mask = iota < (h - tail_lo)

        @plsc.parallel_loop(0, bw, unroll=2)
        def _stage(r):
            rvec = iota * 0 + r
            for j in range(n_full):
                vals = xs[r, pl.ds(j * _L, _L)]
                pos = (j * _L + iota) * bw + r
                plsc.store_scatter(idxb, [pos], vals)
            if h % _L:
                vals = plsc.load_gather(
                    xs, [rvec, tail_lo + iota], mask=tail_mask
                )
                pos = (tail_lo + iota) * bw + r
                plsc.store_scatter(idxb, [pos], vals, mask=tail_mask)

        def g_desc(u, par2):
            off = (u // nbk) * bw + (u % nbk) * 128
            return pltpu.make_async_copy(
                table_hbm.at[idxb.at[pl.ds(off, 128)]], rows[par2], gsems[par2]
            )

        g_desc(0, 0).start()

        @pl.loop(0, n_units // 4)
        def _it(it):
            for par in range(4):
                u = it * 4 + par
                rbuf = rows[par % 2]
                ob = obs[par]
                g_desc(u, par % 2).wait()
                if par < 3:
                    g_desc(u + 1, (par + 1) % 2).start()
                else:
                    @pl.when(it < n_units // 4 - 1)
                    def _():
                        g_desc(u + 1, (par + 1) % 2).start()

                @pl.when(it >= 1)
                def _():
                    for ej in range(et):
                        pltpu.make_async_copy(
                            ob.at[ej],
                            out_hbm.at[it - 1, ej, wid * nbk + par],
                            wsems[par],
                        ).wait()

                @plsc.parallel_loop(0, d, unroll=2)
                def _col(v):
                    col = iota * 0 + v
                    ej, s = v // 8, v % 8
                    for r0 in range(0, 128, _L):
                        vals = plsc.load_gather(rbuf, [r0 + iota, col])
                        ob[ej, s, pl.ds(r0, _L)] = vals * scale

                for ej in range(et):
                    pltpu.async_copy(
                        ob.at[ej], out_hbm.at[it, ej, wid * nbk + par],
                        wsems[par],
                    )

        for par in range(4):
            for ej in range(et):
                pltpu.make_async_copy(
                    obs[par].at[ej],
                    out_hbm.at[n_units // 4 - 1, ej, wid * nbk + par],
                    wsems[par],
                ).wait()

    raw = body(x, table)
    return raw.transpose(2, 4, 0, 1, 3).reshape(b, h, d)


def kernel(x, table):
    return _embed_lookup(x, table)


# speedup vs baseline: 2.6969x; 1.0056x over previous
import functools
import math

import jax
import jax.numpy as jnp
from jax import lax
from jax.experimental import pallas as pl
from jax.experimental.pallas import tpu as pltpu
from jax.experimental.pallas import tpu_sc as plsc

_INFO = plsc.get_sparse_core_info()
_NC, _NS, _L = _INFO.num_cores, _INFO.num_subcores, _INFO.num_lanes
_NW = _NC * _NS


@jax.jit
def _embed_lookup(x, table):
    b, h = x.shape
    _, d = table.shape
    scale = math.sqrt(d)
    bw = b // _NW
    nbk = bw // 128
    n_units = h * nbk
    hp = (h + 7) // 8 * 8
    et = d // 8
    mesh = plsc.VectorSubcoreMesh(core_axis_name="c", subcore_axis_name="s")

    @functools.partial(
        pl.kernel,
        out_type=jax.ShapeDtypeStruct((h, et, b // 128, 8, 128), jnp.float32),
        mesh=mesh,
        scratch_types=[
            pltpu.VMEM((bw, h), jnp.int32),
            pltpu.VMEM((h * bw,), jnp.int32),
            pltpu.VMEM((128, d), jnp.float32),
            pltpu.VMEM((128, d), jnp.float32),
            pltpu.VMEM((et, 8, 128), jnp.float32),
            pltpu.VMEM((et, 8, 128), jnp.float32),
            pltpu.VMEM((et, 8, 128), jnp.float32),
            pltpu.VMEM((et, 8, 128), jnp.float32),
            pltpu.SemaphoreType.DMA,
            pltpu.SemaphoreType.DMA,
            pltpu.SemaphoreType.DMA,
            pltpu.SemaphoreType.DMA,
            pltpu.SemaphoreType.DMA,
            pltpu.SemaphoreType.DMA,
        ],
        compiler_params=pltpu.CompilerParams(
            use_tc_tiling_on_sc=False, needs_layout_passes=False
        ),
    )
    def body(x_hbm, table_hbm, out_hbm, xs, idxb, r0_v, r1_v,
             o0, o1, o2, o3, g0, g1, w0, w1, w2, w3):
        wid = lax.axis_index("s") * _NC + lax.axis_index("c")
        b0 = wid * bw
        iota = lax.iota(jnp.int32, _L)
        rows = (r0_v, r1_v)
        gsems = (g0, g1)
        obs = (o0, o1, o2, o3)
        wsems = (w0, w1, w2, w3)

        pltpu.sync_copy(x_hbm.at[pl.ds(b0, bw)], xs)

        n_full = h // _L
        tail_lo = n_full * _L
        tail_mask = iota < (h - tail_lo)

        @plsc.parallel_loop(0, bw, unroll=2)
        def _stage(r):
            rvec = iota * 0 + r
            for j in range(n_full):
                vals = xs[r, pl.ds(j * _L, _L)]
                pos = (j * _L + iota) * bw + r
                plsc.store_scatter(idxb, [pos], vals)
            if h % _L:
                vals = plsc.load_gather(
                    xs, [rvec, tail_lo + iota], mask=tail_mask
                )
                pos = (tail_lo + iota) * bw + r
                plsc.store_scatter(idxb, [pos], vals, mask=tail_mask)

        def g_desc(u, par2):
            off = (u // nbk) * bw + (u % nbk) * 128
            return pltpu.make_async_copy(
                table_hbm.at[idxb.at[pl.ds(off, 128)]], rows[par2], gsems[par2]
            )

        g_desc(0, 0).start()

        @pl.loop(0, n_units // 4)
        def _it(it):
            for par in range(4):
                u = it * 4 + par
                rbuf = rows[par % 2]
                ob = obs[par]
                g_desc(u, par % 2).wait()
                if par < 3:
                    g_desc(u + 1, (par + 1) % 2).start()
                else:
                    @pl.when(it < n_units // 4 - 1)
                    def _():
                        g_desc(u + 1, (par + 1) % 2).start()

                @pl.when(it >= 1)
                def _():
                    pltpu.make_async_copy(
                        ob, out_hbm.at[it - 1, pl.ds(0, et), wid * nbk + par],
                        wsems[par],
                    ).wait()

                @plsc.parallel_loop(0, d, unroll=4)
                def _col(v):
                    col = iota * 0 + v
                    ej, s = v // 8, v % 8
                    for r0 in range(0, 128, _L):
                        vals = plsc.load_gather(rbuf, [r0 + iota, col])
                        ob[ej, s, pl.ds(r0, _L)] = vals * scale

                pltpu.async_copy(
                    ob, out_hbm.at[it, pl.ds(0, et), wid * nbk + par],
                    wsems[par],
                )

        for par in range(4):
            pltpu.make_async_copy(
                obs[par],
                out_hbm.at[n_units // 4 - 1, pl.ds(0, et), wid * nbk + par],
                wsems[par],
            ).wait()

    raw = body(x, table)
    return raw.transpose(2, 4, 0, 1, 3).reshape(b, h, d)


def kernel(x, table):
    return _embed_lookup(x, table)
